# baseline (speedup 1.0000x reference)
import jax
import jax.numpy as jnp
from jax import lax
from jax.experimental import pallas as pl
from jax.experimental.pallas import tpu as pltpu
from jax.experimental.pallas import tpu_sc as plsc

B = 16384
D = 64
H = D // 2
NC = 2
NS = 16
NW = NC * NS
BPW = B // NW
CHK = 128
NQ = BPW // CHK

_f32 = jnp.float32


def _issue_rows(idx_ref, q, shift, table, rows, sem):
    def issue(g, _):
        v = lax.shift_right_logical(idx_ref[pl.ds(q * CHK + g * 16, 16)],
                                    shift)
        for l in range(16):
            r = v[l]
            pltpu.async_copy(table.at[pl.ds(r, 1)],
                             rows.at[pl.ds(g * 16 + l, 1)], sem)
        return 0

    lax.fori_loop(0, CHK // 16, issue, 0)


def _gather_body(uidx, iidx, cidx, bidx, ut, it, ct, bt,
                 uo, io, co, bo,
                 uix, iix, cix, bix, ur, ir, cr, br, sem):
    wid = lax.axis_index("s") * NC + lax.axis_index("c")
    base = wid * BPW
    pltpu.sync_copy(uidx.at[wid], uix)
    pltpu.sync_copy(iidx.at[wid], iix)
    pltpu.sync_copy(cidx.at[wid], cix)
    pltpu.sync_copy(bidx.at[wid], bix)
    for q in range(NQ):
        sl = pl.ds(base + q * CHK, CHK)
        _issue_rows(uix, q, 1, ut, ur, sem)
        _issue_rows(iix, q, 1, it, ir, sem)
        _issue_rows(cix, q, 2, ct, cr, sem)
        _issue_rows(bix, q, 2, bt, br, sem)
        pltpu.make_async_copy(ut.at[pl.ds(0, CHK)], ur, sem).wait()
        pltpu.make_async_copy(it.at[pl.ds(0, CHK)], ir, sem).wait()
        pltpu.make_async_copy(ct.at[pl.ds(0, CHK)], cr, sem).wait()
        pltpu.make_async_copy(bt.at[pl.ds(0, CHK)], br, sem).wait()
        pltpu.sync_copy(ur, uo.at[sl])
        pltpu.sync_copy(ir, io.at[sl])
        pltpu.sync_copy(cr, co.at[sl])
        pltpu.sync_copy(br, bo.at[sl])


_gather = pl.kernel(
    _gather_body,
    out_type=(
        jax.ShapeDtypeStruct((B, 128), _f32),
        jax.ShapeDtypeStruct((B, 128), _f32),
        jax.ShapeDtypeStruct((B, 128), _f32),
        jax.ShapeDtypeStruct((B, 128), _f32),
    ),
    mesh=plsc.VectorSubcoreMesh(core_axis_name="c", subcore_axis_name="s",
                                num_cores=NC, num_subcores=NS),
    scratch_types=[
        pltpu.VMEM((BPW,), jnp.int32),
        pltpu.VMEM((BPW,), jnp.int32),
        pltpu.VMEM((BPW,), jnp.int32),
        pltpu.VMEM((BPW,), jnp.int32),
        pltpu.VMEM((CHK, 128), _f32),
        pltpu.VMEM((CHK, 128), _f32),
        pltpu.VMEM((CHK, 128), _f32),
        pltpu.VMEM((CHK, 128), _f32),
        pltpu.SemaphoreType.DMA,
    ],
)

SB = 2048


def _select_body(uix_ref, iix_ref, cix_ref, bix_ref,
                 gu_ref, gi_ref, gc_ref, gb_ref,
                 u_ref, i_ref, c_ref, b_ref):
    gu = gu_ref[...]
    gi = gi_ref[...]
    gc = gc_ref[...]
    gb = gb_ref[...]
    upar = lax.bitwise_and(uix_ref[...], 1)
    ipar = lax.bitwise_and(iix_ref[...], 1)
    cq = lax.bitwise_and(cix_ref[...], 3)
    bq = lax.bitwise_and(bix_ref[...], 3)
    u_ref[...] = jnp.where(upar == 0, gu[:, :D], gu[:, D:])
    i_ref[...] = jnp.where(ipar == 0, gi[:, :D], gi[:, D:])
    c_ref[...] = jnp.where(
        cq == 0, gc[:, :H],
        jnp.where(cq == 1, gc[:, H:2 * H],
                  jnp.where(cq == 2, gc[:, 2 * H:3 * H], gc[:, 3 * H:])))
    b_ref[...] = jnp.where(
        bq == 0, gb[:, :H],
        jnp.where(bq == 1, gb[:, H:2 * H],
                  jnp.where(bq == 2, gb[:, 2 * H:3 * H], gb[:, 3 * H:])))


_select = pl.pallas_call(
    _select_body,
    grid=(B // SB,),
    in_specs=[
        pl.BlockSpec((SB, 1), lambda n: (n, 0)),
        pl.BlockSpec((SB, 1), lambda n: (n, 0)),
        pl.BlockSpec((SB, 1), lambda n: (n, 0)),
        pl.BlockSpec((SB, 1), lambda n: (n, 0)),
        pl.BlockSpec((SB, 128), lambda n: (n, 0)),
        pl.BlockSpec((SB, 128), lambda n: (n, 0)),
        pl.BlockSpec((SB, 128), lambda n: (n, 0)),
        pl.BlockSpec((SB, 128), lambda n: (n, 0)),
    ],
    out_specs=[
        pl.BlockSpec((SB, D), lambda n: (n, 0)),
        pl.BlockSpec((SB, D), lambda n: (n, 0)),
        pl.BlockSpec((SB, H), lambda n: (n, 0)),
        pl.BlockSpec((SB, H), lambda n: (n, 0)),
    ],
    out_shape=[
        jax.ShapeDtypeStruct((B, D), _f32),
        jax.ShapeDtypeStruct((B, D), _f32),
        jax.ShapeDtypeStruct((B, H), _f32),
        jax.ShapeDtypeStruct((B, H), _f32),
    ],
)


def _bn(h, gamma, beta):
    mean = jnp.mean(h, axis=0, keepdims=True)
    var = jnp.mean((h - mean) ** 2, axis=0, keepdims=True)
    return (h - mean) / jnp.sqrt(var + 1e-5) * gamma + beta


def _mlp_body(u_ref, i_ref, c_ref, b_ref, w1u_ref, w1i_ref, w1c_ref, w1b_ref,
              b1_ref, g1_ref, be1_ref, w2_ref, b2_ref, g2_ref, be2_ref,
              w3_ref, b3_ref, o_ref):
    h = (jnp.dot(u_ref[...], w1u_ref[...], preferred_element_type=_f32)
         + jnp.dot(i_ref[...], w1i_ref[...], preferred_element_type=_f32)
         + jnp.dot(c_ref[...], w1c_ref[...], preferred_element_type=_f32)
         + jnp.dot(b_ref[...], w1b_ref[...], preferred_element_type=_f32)
         + b1_ref[...])
    h = jnp.maximum(_bn(h, g1_ref[...], be1_ref[...]), 0.0)
    h = jnp.dot(h, w2_ref[...], preferred_element_type=_f32) + b2_ref[...]
    h = jnp.maximum(_bn(h, g2_ref[...], be2_ref[...]), 0.0)
    out = jnp.dot(h, w3_ref[...], preferred_element_type=_f32) + b3_ref[...]
    o_ref[...] = jax.nn.sigmoid(out)


_mlp = pl.pallas_call(
    _mlp_body,
    out_shape=jax.ShapeDtypeStruct((B, 1), _f32),
)


def kernel(user_idx, item_idx, cat_idx, brand_idx, user_table, item_table,
           cat_table, brand_table, W1, b1, g1, be1, W2, b2, g2, be2, W3, b3):
    uidx = user_idx.astype(jnp.int32)
    iidx = item_idx.astype(jnp.int32)
    cidx = cat_idx.astype(jnp.int32)
    bidx = brand_idx.astype(jnp.int32)
    ut2 = user_table.reshape(-1, 128)
    it2 = item_table.reshape(-1, 128)
    ct2 = cat_table.reshape(-1, 128)
    bt2 = brand_table.reshape(-1, 128)
    gu, gi, gc, gb = _gather(uidx.reshape(NW, BPW), iidx.reshape(NW, BPW),
                             cidx.reshape(NW, BPW), bidx.reshape(NW, BPW),
                             ut2, it2, ct2, bt2)
    u, i, c, b = _select(uidx.reshape(B, 1), iidx.reshape(B, 1),
                         cidx.reshape(B, 1), bidx.reshape(B, 1),
                         gu, gi, gc, gb)
    w1u = W1[:D]
    w1i = W1[D:2 * D]
    w1c = W1[2 * D:2 * D + H]
    w1b = W1[2 * D + H:]
    b1r = b1.reshape(1, -1)
    g1r = g1.reshape(1, -1)
    be1r = be1.reshape(1, -1)
    b2r = b2.reshape(1, -1)
    g2r = g2.reshape(1, -1)
    be2r = be2.reshape(1, -1)
    b3r = b3.reshape(1, -1)
    out = _mlp(u, i, c, b, w1u, w1i, w1c, w1b, b1r, g1r, be1r,
               W2, b2r, g2r, be2r, W3, b3r)
    return jnp.squeeze(out, axis=-1)


# speedup vs baseline: 1.1240x; 1.1240x over previous
import jax
import jax.numpy as jnp
from jax import lax
from jax.experimental import pallas as pl
from jax.experimental.pallas import tpu as pltpu
from jax.experimental.pallas import tpu_sc as plsc

B = 16384
D = 64
H = D // 2
NC = 2
NS = 16
NW = NC * NS
BPW = B // NW
CHK = 128
NQ = BPW // CHK

_f32 = jnp.float32
_u32 = jnp.uint32


def _make_relayout(W, N, BLK):
    grid = (N + BLK - 1) // BLK
    W2 = W // 2

    def body(p_ref, o_ref):
        x = p_ref[...]
        f = lax.broadcasted_iota(jnp.int32, (W, W2), 0)
        j = lax.broadcasted_iota(jnp.int32, (W, W2), 1)
        ee = (f == 2 * j).astype(_f32)
        eo = (f == 2 * j + 1).astype(_f32)
        dn = (((0,), (0,)), ((), ()))
        xe = lax.dot_general(x, ee, dn, preferred_element_type=_f32)
        xo = lax.dot_general(x, eo, dn, preferred_element_type=_f32)
        be = lax.bitcast_convert_type(
            xe.astype(jnp.bfloat16).astype(_f32), _u32)
        bo = lax.bitcast_convert_type(
            xo.astype(jnp.bfloat16).astype(_f32), _u32)
        o_ref[...] = (lax.shift_right_logical(be, jnp.uint32(16))
                      | lax.bitwise_and(bo, jnp.uint32(0xFFFF0000)))

    return pl.pallas_call(
        body,
        grid=grid,
        in_specs=[pl.BlockSpec((W, BLK), lambda n: (0, n))],
        out_specs=pl.BlockSpec((BLK, W2), lambda n: (n, 0)),
        out_shape=jax.ShapeDtypeStruct((N, W2), _u32),
    )


_relay_user = _make_relayout(D, 1000000, 2048)
_relay_item = _make_relayout(D, 1000000, 2048)
_relay_cat = _make_relayout(H, 1000, 1024)
_relay_brand = _make_relayout(H, 100000, 4096)


def _issue_rows(idx_ref, q, table, rows, sem):
    def issue(g, _):
        v = idx_ref[pl.ds(q * CHK + g * 16, 16)]
        for l in range(16):
            r = v[l]
            pltpu.async_copy(table.at[pl.ds(r, 1)],
                             rows.at[pl.ds(g * 16 + l, 1)], sem)
        return 0

    lax.fori_loop(0, CHK // 16, issue, 0)


def _gather_body(uidx, iidx, cidx, bidx, ut, it, ct, bt,
                 uo, io, co, bo,
                 uix, iix, cix, bix, ur, ir, cr, br, sem):
    wid = lax.axis_index("s") * NC + lax.axis_index("c")
    base = wid * BPW
    pltpu.sync_copy(uidx.at[wid], uix)
    pltpu.sync_copy(iidx.at[wid], iix)
    pltpu.sync_copy(cidx.at[wid], cix)
    pltpu.sync_copy(bidx.at[wid], bix)
    for q in range(NQ):
        sl = pl.ds(base + q * CHK, CHK)
        _issue_rows(uix, q, ut, ur, sem)
        _issue_rows(iix, q, it, ir, sem)
        _issue_rows(cix, q, ct, cr, sem)
        _issue_rows(bix, q, bt, br, sem)
        pltpu.make_async_copy(ut.at[pl.ds(0, CHK)], ur, sem).wait()
        pltpu.make_async_copy(it.at[pl.ds(0, CHK)], ir, sem).wait()
        pltpu.make_async_copy(ct.at[pl.ds(0, CHK)], cr, sem).wait()
        pltpu.make_async_copy(bt.at[pl.ds(0, CHK)], br, sem).wait()
        pltpu.sync_copy(ur, uo.at[sl])
        pltpu.sync_copy(ir, io.at[sl])
        pltpu.sync_copy(cr, co.at[sl])
        pltpu.sync_copy(br, bo.at[sl])


_gather = pl.kernel(
    _gather_body,
    out_type=(
        jax.ShapeDtypeStruct((B, H), _u32),
        jax.ShapeDtypeStruct((B, H), _u32),
        jax.ShapeDtypeStruct((B, H // 2), _u32),
        jax.ShapeDtypeStruct((B, H // 2), _u32),
    ),
    mesh=plsc.VectorSubcoreMesh(core_axis_name="c", subcore_axis_name="s",
                                num_cores=NC, num_subcores=NS),
    scratch_types=[
        pltpu.VMEM((BPW,), jnp.int32),
        pltpu.VMEM((BPW,), jnp.int32),
        pltpu.VMEM((BPW,), jnp.int32),
        pltpu.VMEM((BPW,), jnp.int32),
        pltpu.VMEM((CHK, H), _u32),
        pltpu.VMEM((CHK, H), _u32),
        pltpu.VMEM((CHK, H // 2), _u32),
        pltpu.VMEM((CHK, H // 2), _u32),
        pltpu.SemaphoreType.DMA,
    ],
)


def _unpack(x):
    lo = lax.bitcast_convert_type(lax.shift_left(x, jnp.uint32(16)), _f32)
    hi = lax.bitcast_convert_type(lax.bitwise_and(x, jnp.uint32(0xFFFF0000)), _f32)
    return lo, hi


def _bn(h, gamma, beta):
    mean = jnp.mean(h, axis=0, keepdims=True)
    var = jnp.mean((h - mean) ** 2, axis=0, keepdims=True)
    return (h - mean) / jnp.sqrt(var + 1e-5) * gamma + beta


def _mlp_body(u_ref, i_ref, c_ref, b_ref,
              wue_ref, wuo_ref, wie_ref, wio_ref,
              wce_ref, wco_ref, wbe_ref, wbo_ref,
              b1_ref, g1_ref, be1_ref, w2_ref, b2_ref, g2_ref, be2_ref,
              w3_ref, b3_ref, o_ref):
    ue, uo = _unpack(u_ref[...])
    ie, io = _unpack(i_ref[...])
    ce, co = _unpack(c_ref[...])
    be, bo = _unpack(b_ref[...])
    h = (jnp.dot(ue, wue_ref[...], preferred_element_type=_f32)
         + jnp.dot(uo, wuo_ref[...], preferred_element_type=_f32)
         + jnp.dot(ie, wie_ref[...], preferred_element_type=_f32)
         + jnp.dot(io, wio_ref[...], preferred_element_type=_f32)
         + jnp.dot(ce, wce_ref[...], preferred_element_type=_f32)
         + jnp.dot(co, wco_ref[...], preferred_element_type=_f32)
         + jnp.dot(be, wbe_ref[...], preferred_element_type=_f32)
         + jnp.dot(bo, wbo_ref[...], preferred_element_type=_f32)
         + b1_ref[...])
    h = jnp.maximum(_bn(h, g1_ref[...], be1_ref[...]), 0.0)
    h = jnp.dot(h, w2_ref[...], preferred_element_type=_f32) + b2_ref[...]
    h = jnp.maximum(_bn(h, g2_ref[...], be2_ref[...]), 0.0)
    out = jnp.dot(h, w3_ref[...], preferred_element_type=_f32) + b3_ref[...]
    o_ref[...] = jax.nn.sigmoid(out)


_mlp = pl.pallas_call(
    _mlp_body,
    out_shape=jax.ShapeDtypeStruct((B, 1), _f32),
    compiler_params=pltpu.CompilerParams(vmem_limit_bytes=100 * 1024 * 1024),
)


def kernel(user_idx, item_idx, cat_idx, brand_idx, user_table, item_table,
           cat_table, brand_table, W1, b1, g1, be1, W2, b2, g2, be2, W3, b3):
    uidx = user_idx.astype(jnp.int32).reshape(NW, BPW)
    iidx = item_idx.astype(jnp.int32).reshape(NW, BPW)
    cidx = cat_idx.astype(jnp.int32).reshape(NW, BPW)
    bidx = brand_idx.astype(jnp.int32).reshape(NW, BPW)
    ut2 = _relay_user(user_table.T)
    it2 = _relay_item(item_table.T)
    ct2 = _relay_cat(cat_table.T)
    bt2 = _relay_brand(brand_table.T)
    u, i, c, b = _gather(uidx, iidx, cidx, bidx, ut2, it2, ct2, bt2)
    w1u = W1[:D]
    w1i = W1[D:2 * D]
    w1c = W1[2 * D:2 * D + H]
    w1b = W1[2 * D + H:]
    b1r = b1.reshape(1, -1)
    g1r = g1.reshape(1, -1)
    be1r = be1.reshape(1, -1)
    b2r = b2.reshape(1, -1)
    g2r = g2.reshape(1, -1)
    be2r = be2.reshape(1, -1)
    b3r = b3.reshape(1, -1)
    out = _mlp(u, i, c, b,
               w1u[0::2], w1u[1::2], w1i[0::2], w1i[1::2],
               w1c[0::2], w1c[1::2], w1b[0::2], w1b[1::2],
               b1r, g1r, be1r, W2, b2r, g2r, be2r, W3, b3r)
    return jnp.squeeze(out, axis=-1)


# speedup vs baseline: 1.1243x; 1.0003x over previous
import jax
import jax.numpy as jnp
from jax import lax
from jax.experimental import pallas as pl
from jax.experimental.pallas import tpu as pltpu
from jax.experimental.pallas import tpu_sc as plsc

B = 16384
D = 64
H = D // 2
NC = 2
NS = 16
NW = NC * NS
BPW = B // NW
CHK = 128
NQ = BPW // CHK

_f32 = jnp.float32
_u32 = jnp.uint32


def _make_relayout(W, N, BLK):
    grid = (N + BLK - 1) // BLK
    W2 = W // 2

    def body(p_ref, o_ref):
        x = p_ref[...]
        f = lax.broadcasted_iota(jnp.int32, (W, W2), 0)
        j = lax.broadcasted_iota(jnp.int32, (W, W2), 1)
        ee = (f == 2 * j).astype(_f32)
        eo = (f == 2 * j + 1).astype(_f32)
        dn = (((0,), (0,)), ((), ()))
        xe = lax.dot_general(x, ee, dn, preferred_element_type=_f32)
        xo = lax.dot_general(x, eo, dn, preferred_element_type=_f32)
        be = lax.bitcast_convert_type(
            xe.astype(jnp.bfloat16).astype(_f32), _u32)
        bo = lax.bitcast_convert_type(
            xo.astype(jnp.bfloat16).astype(_f32), _u32)
        o_ref[...] = (lax.shift_right_logical(be, jnp.uint32(16))
                      | lax.bitwise_and(bo, jnp.uint32(0xFFFF0000)))

    return pl.pallas_call(
        body,
        grid=grid,
        in_specs=[pl.BlockSpec((W, BLK), lambda n: (0, n))],
        out_specs=pl.BlockSpec((BLK, W2), lambda n: (n, 0)),
        out_shape=jax.ShapeDtypeStruct((N, W2), _u32),
        compiler_params=pltpu.CompilerParams(
            fuse_transposed_lhs_in_matmul=True),
    )


_relay_user = _make_relayout(D, 1000000, 2048)
_relay_item = _make_relayout(D, 1000000, 2048)
_relay_cat = _make_relayout(H, 1000, 1024)
_relay_brand = _make_relayout(H, 100000, 4096)


def _issue_rows(idx_ref, q, table, rows, sem):
    def issue(g, _):
        v = idx_ref[pl.ds(q * CHK + g * 16, 16)]
        for l in range(16):
            r = v[l]
            pltpu.async_copy(table.at[pl.ds(r, 1)],
                             rows.at[pl.ds(g * 16 + l, 1)], sem)
        return 0

    lax.fori_loop(0, CHK // 16, issue, 0)


def _gather_body(uidx, iidx, cidx, bidx, ut, it, ct, bt,
                 uo, io, co, bo,
                 uix, iix, cix, bix, ur, ir, cr, br, sem):
    wid = lax.axis_index("s") * NC + lax.axis_index("c")
    base = wid * BPW
    pltpu.sync_copy(uidx.at[wid], uix)
    pltpu.sync_copy(iidx.at[wid], iix)
    pltpu.sync_copy(cidx.at[wid], cix)
    pltpu.sync_copy(bidx.at[wid], bix)
    for q in range(NQ):
        sl = pl.ds(base + q * CHK, CHK)
        _issue_rows(uix, q, ut, ur, sem)
        _issue_rows(iix, q, it, ir, sem)
        _issue_rows(cix, q, ct, cr, sem)
        _issue_rows(bix, q, bt, br, sem)
        pltpu.make_async_copy(ut.at[pl.ds(0, CHK)], ur, sem).wait()
        pltpu.make_async_copy(it.at[pl.ds(0, CHK)], ir, sem).wait()
        pltpu.make_async_copy(ct.at[pl.ds(0, CHK)], cr, sem).wait()
        pltpu.make_async_copy(bt.at[pl.ds(0, CHK)], br, sem).wait()
        pltpu.sync_copy(ur, uo.at[sl])
        pltpu.sync_copy(ir, io.at[sl])
        pltpu.sync_copy(cr, co.at[sl])
        pltpu.sync_copy(br, bo.at[sl])


_gather = pl.kernel(
    _gather_body,
    out_type=(
        jax.ShapeDtypeStruct((B, H), _u32),
        jax.ShapeDtypeStruct((B, H), _u32),
        jax.ShapeDtypeStruct((B, H // 2), _u32),
        jax.ShapeDtypeStruct((B, H // 2), _u32),
    ),
    mesh=plsc.VectorSubcoreMesh(core_axis_name="c", subcore_axis_name="s",
                                num_cores=NC, num_subcores=NS),
    scratch_types=[
        pltpu.VMEM((BPW,), jnp.int32),
        pltpu.VMEM((BPW,), jnp.int32),
        pltpu.VMEM((BPW,), jnp.int32),
        pltpu.VMEM((BPW,), jnp.int32),
        pltpu.VMEM((CHK, H), _u32),
        pltpu.VMEM((CHK, H), _u32),
        pltpu.VMEM((CHK, H // 2), _u32),
        pltpu.VMEM((CHK, H // 2), _u32),
        pltpu.SemaphoreType.DMA,
    ],
)


def _unpack(x):
    lo = lax.bitcast_convert_type(lax.shift_left(x, jnp.uint32(16)), _f32)
    hi = lax.bitcast_convert_type(lax.bitwise_and(x, jnp.uint32(0xFFFF0000)), _f32)
    return lo, hi


def _bn(h, gamma, beta):
    mean = jnp.mean(h, axis=0, keepdims=True)
    var = jnp.mean((h - mean) ** 2, axis=0, keepdims=True)
    return (h - mean) / jnp.sqrt(var + 1e-5) * gamma + beta


def _mlp_body(u_ref, i_ref, c_ref, b_ref,
              wue_ref, wuo_ref, wie_ref, wio_ref,
              wce_ref, wco_ref, wbe_ref, wbo_ref,
              b1_ref, g1_ref, be1_ref, w2_ref, b2_ref, g2_ref, be2_ref,
              w3_ref, b3_ref, o_ref):
    ue, uo = _unpack(u_ref[...])
    ie, io = _unpack(i_ref[...])
    ce, co = _unpack(c_ref[...])
    be, bo = _unpack(b_ref[...])
    h = (jnp.dot(ue, wue_ref[...], preferred_element_type=_f32)
         + jnp.dot(uo, wuo_ref[...], preferred_element_type=_f32)
         + jnp.dot(ie, wie_ref[...], preferred_element_type=_f32)
         + jnp.dot(io, wio_ref[...], preferred_element_type=_f32)
         + jnp.dot(ce, wce_ref[...], preferred_element_type=_f32)
         + jnp.dot(co, wco_ref[...], preferred_element_type=_f32)
         + jnp.dot(be, wbe_ref[...], preferred_element_type=_f32)
         + jnp.dot(bo, wbo_ref[...], preferred_element_type=_f32)
         + b1_ref[...])
    h = jnp.maximum(_bn(h, g1_ref[...], be1_ref[...]), 0.0)
    h = jnp.dot(h, w2_ref[...], preferred_element_type=_f32) + b2_ref[...]
    h = jnp.maximum(_bn(h, g2_ref[...], be2_ref[...]), 0.0)
    out = jnp.dot(h, w3_ref[...], preferred_element_type=_f32) + b3_ref[...]
    o_ref[...] = jax.nn.sigmoid(out)


_mlp = pl.pallas_call(
    _mlp_body,
    out_shape=jax.ShapeDtypeStruct((B, 1), _f32),
    compiler_params=pltpu.CompilerParams(vmem_limit_bytes=100 * 1024 * 1024),
)


def kernel(user_idx, item_idx, cat_idx, brand_idx, user_table, item_table,
           cat_table, brand_table, W1, b1, g1, be1, W2, b2, g2, be2, W3, b3):
    uidx = user_idx.astype(jnp.int32).reshape(NW, BPW)
    iidx = item_idx.astype(jnp.int32).reshape(NW, BPW)
    cidx = cat_idx.astype(jnp.int32).reshape(NW, BPW)
    bidx = brand_idx.astype(jnp.int32).reshape(NW, BPW)
    ut2 = _relay_user(user_table.T)
    it2 = _relay_item(item_table.T)
    ct2 = _relay_cat(cat_table.T)
    bt2 = _relay_brand(brand_table.T)
    u, i, c, b = _gather(uidx, iidx, cidx, bidx, ut2, it2, ct2, bt2)
    w1u = W1[:D]
    w1i = W1[D:2 * D]
    w1c = W1[2 * D:2 * D + H]
    w1b = W1[2 * D + H:]
    b1r = b1.reshape(1, -1)
    g1r = g1.reshape(1, -1)
    be1r = be1.reshape(1, -1)
    b2r = b2.reshape(1, -1)
    g2r = g2.reshape(1, -1)
    be2r = be2.reshape(1, -1)
    b3r = b3.reshape(1, -1)
    out = _mlp(u, i, c, b,
               w1u[0::2], w1u[1::2], w1i[0::2], w1i[1::2],
               w1c[0::2], w1c[1::2], w1b[0::2], w1b[1::2],
               b1r, g1r, be1r, W2, b2r, g2r, be2r, W3, b3r)
    return jnp.squeeze(out, axis=-1)


# speedup vs baseline: 1.5805x; 1.4057x over previous
import jax
import jax.numpy as jnp
from jax import lax
from jax.experimental import pallas as pl
from jax.experimental.pallas import tpu as pltpu
from jax.experimental.pallas import tpu_sc as plsc

B = 16384
D = 64
H = D // 2
NC = 2
NS = 16
NW = NC * NS
BPW = B // NW
CHK = 128
NQ = BPW // CHK

_f32 = jnp.float32


def _issue_rows(idx_ref, q, table, rows, sem):
    def issue(g, _):
        v = idx_ref[pl.ds(q * CHK + g * 16, 16)]
        for l in range(16):
            r = v[l]
            pltpu.async_copy(table.at[pl.ds(r, 1)],
                             rows.at[pl.ds(g * 16 + l, 1)], sem)
        return 0

    lax.fori_loop(0, CHK // 16, issue, 0)


def _gather_body(uidx, iidx, cidx, bidx, ut, it, ct, bt,
                 uo, io, co, bo,
                 uix, iix, cix, bix, ur, ir, cr, br, sem):
    wid = lax.axis_index("s") * NC + lax.axis_index("c")
    base = wid * BPW
    pltpu.sync_copy(uidx.at[wid], uix)
    pltpu.sync_copy(iidx.at[wid], iix)
    pltpu.sync_copy(cidx.at[wid], cix)
    pltpu.sync_copy(bidx.at[wid], bix)
    for q in range(NQ):
        sl = pl.ds(base + q * CHK, CHK)
        _issue_rows(uix, q, ut, ur, sem)
        _issue_rows(iix, q, it, ir, sem)
        _issue_rows(cix, q, ct, cr, sem)
        _issue_rows(bix, q, bt, br, sem)
        pltpu.make_async_copy(ut.at[pl.ds(0, CHK)], ur, sem).wait()
        pltpu.make_async_copy(it.at[pl.ds(0, CHK)], ir, sem).wait()
        pltpu.make_async_copy(ct.at[pl.ds(0, CHK)], cr, sem).wait()
        pltpu.make_async_copy(bt.at[pl.ds(0, CHK)], br, sem).wait()
        pltpu.sync_copy(ur, uo.at[sl])
        pltpu.sync_copy(ir, io.at[sl])
        pltpu.sync_copy(cr, co.at[sl])
        pltpu.sync_copy(br, bo.at[sl])


_gather = pl.kernel(
    _gather_body,
    out_type=(
        jax.ShapeDtypeStruct((B, D), _f32),
        jax.ShapeDtypeStruct((B, D), _f32),
        jax.ShapeDtypeStruct((B, H), _f32),
        jax.ShapeDtypeStruct((B, H), _f32),
    ),
    mesh=plsc.VectorSubcoreMesh(core_axis_name="c", subcore_axis_name="s",
                                num_cores=NC, num_subcores=NS),
    scratch_types=[
        pltpu.VMEM((BPW,), jnp.int32),
        pltpu.VMEM((BPW,), jnp.int32),
        pltpu.VMEM((BPW,), jnp.int32),
        pltpu.VMEM((BPW,), jnp.int32),
        pltpu.VMEM((CHK, D), _f32),
        pltpu.VMEM((CHK, D), _f32),
        pltpu.VMEM((CHK, H), _f32),
        pltpu.VMEM((CHK, H), _f32),
        pltpu.SemaphoreType.DMA,
    ],
)


def _bn(h, gamma, beta):
    mean = jnp.mean(h, axis=0, keepdims=True)
    var = jnp.mean((h - mean) ** 2, axis=0, keepdims=True)
    return (h - mean) / jnp.sqrt(var + 1e-5) * gamma + beta


def _mlp_body(u_ref, i_ref, c_ref, b_ref, w1u_ref, w1i_ref, w1c_ref, w1b_ref,
              b1_ref, g1_ref, be1_ref, w2_ref, b2_ref, g2_ref, be2_ref,
              w3_ref, b3_ref, o_ref):
    h = (jnp.dot(u_ref[...], w1u_ref[...], preferred_element_type=_f32)
         + jnp.dot(i_ref[...], w1i_ref[...], preferred_element_type=_f32)
         + jnp.dot(c_ref[...], w1c_ref[...], preferred_element_type=_f32)
         + jnp.dot(b_ref[...], w1b_ref[...], preferred_element_type=_f32)
         + b1_ref[...])
    h = jnp.maximum(_bn(h, g1_ref[...], be1_ref[...]), 0.0)
    h = jnp.dot(h, w2_ref[...], preferred_element_type=_f32) + b2_ref[...]
    h = jnp.maximum(_bn(h, g2_ref[...], be2_ref[...]), 0.0)
    out = jnp.dot(h, w3_ref[...], preferred_element_type=_f32) + b3_ref[...]
    o_ref[...] = jax.nn.sigmoid(out)


_mlp = pl.pallas_call(
    _mlp_body,
    out_shape=jax.ShapeDtypeStruct((B, 1), _f32),
)


def kernel(user_idx, item_idx, cat_idx, brand_idx, user_table, item_table,
           cat_table, brand_table, W1, b1, g1, be1, W2, b2, g2, be2, W3, b3):
    uidx = user_idx.astype(jnp.int32).reshape(NW, BPW)
    iidx = item_idx.astype(jnp.int32).reshape(NW, BPW)
    cidx = cat_idx.astype(jnp.int32).reshape(NW, BPW)
    bidx = brand_idx.astype(jnp.int32).reshape(NW, BPW)
    u, i, c, b = _gather(uidx, iidx, cidx, bidx,
                         user_table, item_table, cat_table, brand_table)
    w1u = W1[:D]
    w1i = W1[D:2 * D]
    w1c = W1[2 * D:2 * D + H]
    w1b = W1[2 * D + H:]
    b1r = b1.reshape(1, -1)
    g1r = g1.reshape(1, -1)
    be1r = be1.reshape(1, -1)
    b2r = b2.reshape(1, -1)
    g2r = g2.reshape(1, -1)
    be2r = be2.reshape(1, -1)
    b3r = b3.reshape(1, -1)
    out = _mlp(u, i, c, b, w1u, w1i, w1c, w1b, b1r, g1r, be1r,
               W2, b2r, g2r, be2r, W3, b3r)
    return jnp.squeeze(out, axis=-1)


# speedup vs baseline: 1.5907x; 1.0064x over previous
import jax
import jax.numpy as jnp
from jax import lax
from jax.experimental import pallas as pl
from jax.experimental.pallas import tpu as pltpu
from jax.experimental.pallas import tpu_sc as plsc

B = 16384
D = 64
H = D // 2
NC = 2
NS = 16
NW = NC * NS
BPW = B // NW
CHK = 128
NQ = BPW // CHK

_f32 = jnp.float32


def _issue_rows(idx_ref, q, table, rows, sem):
    def issue(g, _):
        v = idx_ref[pl.ds(q * CHK + g * 16, 16)]
        for l in range(16):
            r = v[l]
            pltpu.async_copy(table.at[pl.ds(r, 1)],
                             rows.at[pl.ds(g * 16 + l, 1)], sem)
        return 0

    lax.fori_loop(0, CHK // 16, issue, 0)


def _gather3_body(iidx, cidx, bidx, it, ct, bt,
                  io, co, bo,
                  iix, cix, bix, ir, cr, br, sem):
    wid = lax.axis_index("s") * NC + lax.axis_index("c")
    base = wid * BPW
    pltpu.sync_copy(iidx.at[wid], iix)
    pltpu.sync_copy(cidx.at[wid], cix)
    pltpu.sync_copy(bidx.at[wid], bix)
    for q in range(NQ):
        sl = pl.ds(base + q * CHK, CHK)
        _issue_rows(iix, q, it, ir, sem)
        _issue_rows(cix, q, ct, cr, sem)
        _issue_rows(bix, q, bt, br, sem)
        pltpu.make_async_copy(it.at[pl.ds(0, CHK)], ir, sem).wait()
        pltpu.make_async_copy(ct.at[pl.ds(0, CHK)], cr, sem).wait()
        pltpu.make_async_copy(bt.at[pl.ds(0, CHK)], br, sem).wait()
        pltpu.sync_copy(ir, io.at[sl])
        pltpu.sync_copy(cr, co.at[sl])
        pltpu.sync_copy(br, bo.at[sl])


_gather3 = pl.kernel(
    _gather3_body,
    out_type=(
        jax.ShapeDtypeStruct((B, D), _f32),
        jax.ShapeDtypeStruct((B, H), _f32),
        jax.ShapeDtypeStruct((B, H), _f32),
    ),
    mesh=plsc.VectorSubcoreMesh(core_axis_name="c", subcore_axis_name="s",
                                num_cores=NC, num_subcores=NS),
    scratch_types=[
        pltpu.VMEM((BPW,), jnp.int32),
        pltpu.VMEM((BPW,), jnp.int32),
        pltpu.VMEM((BPW,), jnp.int32),
        pltpu.VMEM((CHK, D), _f32),
        pltpu.VMEM((CHK, H), _f32),
        pltpu.VMEM((CHK, H), _f32),
        pltpu.SemaphoreType.DMA,
    ],
)


def _gather1_body(uidx, ut, uo, uix, ur, sem):
    wid = lax.axis_index("s") * NC + lax.axis_index("c")
    base = wid * BPW
    pltpu.sync_copy(uidx.at[wid], uix)
    for q in range(NQ):
        sl = pl.ds(base + q * CHK, CHK)
        _issue_rows(uix, q, ut, ur, sem)
        pltpu.make_async_copy(ut.at[pl.ds(0, CHK)], ur, sem).wait()
        pltpu.sync_copy(ur, uo.at[sl])


_gather1 = pl.kernel(
    _gather1_body,
    out_type=jax.ShapeDtypeStruct((B, D), _f32),
    mesh=plsc.VectorSubcoreMesh(core_axis_name="c", subcore_axis_name="s",
                                num_cores=NC, num_subcores=NS),
    scratch_types=[
        pltpu.VMEM((BPW,), jnp.int32),
        pltpu.VMEM((CHK, D), _f32),
        pltpu.SemaphoreType.DMA,
    ],
)


def _bn(h, gamma, beta):
    mean = jnp.mean(h, axis=0, keepdims=True)
    var = jnp.mean((h - mean) ** 2, axis=0, keepdims=True)
    return (h - mean) / jnp.sqrt(var + 1e-5) * gamma + beta


def _mlp_body(u_ref, i_ref, c_ref, b_ref, w1u_ref, w1i_ref, w1c_ref, w1b_ref,
              b1_ref, g1_ref, be1_ref, w2_ref, b2_ref, g2_ref, be2_ref,
              w3_ref, b3_ref, o_ref):
    h = (jnp.dot(u_ref[...], w1u_ref[...], preferred_element_type=_f32)
         + jnp.dot(i_ref[...], w1i_ref[...], preferred_element_type=_f32)
         + jnp.dot(c_ref[...], w1c_ref[...], preferred_element_type=_f32)
         + jnp.dot(b_ref[...], w1b_ref[...], preferred_element_type=_f32)
         + b1_ref[...])
    h = jnp.maximum(_bn(h, g1_ref[...], be1_ref[...]), 0.0)
    h = jnp.dot(h, w2_ref[...], preferred_element_type=_f32) + b2_ref[...]
    h = jnp.maximum(_bn(h, g2_ref[...], be2_ref[...]), 0.0)
    out = jnp.dot(h, w3_ref[...], preferred_element_type=_f32) + b3_ref[...]
    o_ref[...] = jax.nn.sigmoid(out)


_mlp = pl.pallas_call(
    _mlp_body,
    out_shape=jax.ShapeDtypeStruct((B, 1), _f32),
)


def kernel(user_idx, item_idx, cat_idx, brand_idx, user_table, item_table,
           cat_table, brand_table, W1, b1, g1, be1, W2, b2, g2, be2, W3, b3):
    uidx = user_idx.astype(jnp.int32).reshape(NW, BPW)
    iidx = item_idx.astype(jnp.int32).reshape(NW, BPW)
    cidx = cat_idx.astype(jnp.int32).reshape(NW, BPW)
    bidx = brand_idx.astype(jnp.int32).reshape(NW, BPW)
    i, c, b = _gather3(iidx, cidx, bidx, item_table, cat_table, brand_table)
    u = _gather1(uidx, user_table)
    w1u = W1[:D]
    w1i = W1[D:2 * D]
    w1c = W1[2 * D:2 * D + H]
    w1b = W1[2 * D + H:]
    b1r = b1.reshape(1, -1)
    g1r = g1.reshape(1, -1)
    be1r = be1.reshape(1, -1)
    b2r = b2.reshape(1, -1)
    g2r = g2.reshape(1, -1)
    be2r = be2.reshape(1, -1)
    b3r = b3.reshape(1, -1)
    out = _mlp(u, i, c, b, w1u, w1i, w1c, w1b, b1r, g1r, be1r,
               W2, b2r, g2r, be2r, W3, b3r)
    return jnp.squeeze(out, axis=-1)
